# Initial kernel scaffold; baseline (speedup 1.0000x reference)
import jax
import jax.numpy as jnp
from jax.experimental import pallas as pl


def kernel(edge_index, edge_vals, embeds):
    raise NotImplementedError("write your pallas kernel here")


# speedup vs baseline: 3.0297x; 3.0297x over previous
import jax
import jax.numpy as jnp
from jax import lax
from jax.experimental import pallas as pl
from jax.experimental.pallas import tpu as pltpu
from jax.experimental.pallas import tpu_sc as plsc

N_NODES = 10000
N_EDGES = 320000
D_FEAT = 128
NLANE = D_FEAT // 16

NC = 2
NS = 16
NW = NC * NS

N_PAD = 10240
SUB = 32
SPC = 8
CH = SUB * SPC
NCH = 40
EPW = CH * NCH
E_PAD = EPW * NW
IDX_ROWS = E_PAD // SUB
RPW = EPW // SUB
ZROWS = 16
RPT = N_PAD // NS


def _sc_body(src_r, dst_r, vals, embeds, parts,
             acc, src_buf, dst_buf, val_buf, rows_buf, zero_buf, sem):
    c = lax.axis_index("c")
    s = lax.axis_index("s")
    wid = s * NC + c

    zvec = jnp.zeros((16,), jnp.float32)
    for r in range(ZROWS):
        for d in range(NLANE):
            zero_buf[r, pl.ds(d * 16, 16)] = zvec
    for i in range(RPT // ZROWS):
        pltpu.sync_copy(zero_buf, acc.at[pl.ds(s * RPT + i * ZROWS, ZROWS)])
    plsc.subcore_barrier()

    row0 = wid * RPW
    e0 = wid * EPW

    def chunk_body(i, carry):
        rb = row0 + i * SPC
        eb = e0 + i * CH
        pltpu.sync_copy(src_r.at[pl.ds(rb, SPC)], src_buf)
        pltpu.sync_copy(dst_r.at[pl.ds(rb, SPC)], dst_buf)
        pltpu.sync_copy(vals.at[pl.ds(eb, CH)], val_buf)
        cps = [
            pltpu.async_copy(embeds.at[src_buf.at[j]], rows_buf.at[j], sem)
            for j in range(SPC)
        ]
        for cp in cps:
            cp.wait()

        for j in range(SPC):
            def mul_body(g, carry2, j=j):
                off = pl.multiple_of(g * 16, 16)
                v16 = val_buf[pl.ds(j * SUB + off, 16)]
                for k in range(16):
                    bv = jnp.full((16,), v16[k])
                    r = off + k
                    for d in range(NLANE):
                        rows_buf[j, r, pl.ds(d * 16, 16)] = (
                            rows_buf[j, r, pl.ds(d * 16, 16)] * bv
                        )
                return carry2

            lax.fori_loop(0, SUB // 16, mul_body, 0)

        for j in range(SPC):
            pltpu.sync_copy(rows_buf.at[j], acc.at[dst_buf.at[j]], add=True)
        return carry

    lax.fori_loop(0, NCH, chunk_body, 0)
    plsc.subcore_barrier()
    pltpu.sync_copy(
        acc.at[pl.ds(s * RPT, RPT)],
        parts.at[c, pl.ds(s * RPT, RPT)],
    )


_sc_spmm = pl.kernel(
    _sc_body,
    out_type=jax.ShapeDtypeStruct((NC, N_PAD, D_FEAT), jnp.float32),
    mesh=plsc.VectorSubcoreMesh(core_axis_name="c", subcore_axis_name="s"),
    scratch_types=[
        pltpu.VMEM_SHARED((N_PAD, D_FEAT), jnp.float32),
        pltpu.VMEM((SPC, SUB), jnp.int32),
        pltpu.VMEM((SPC, SUB), jnp.int32),
        pltpu.VMEM((CH,), jnp.float32),
        pltpu.VMEM((SPC, SUB, D_FEAT), jnp.float32),
        pltpu.VMEM((ZROWS, D_FEAT), jnp.float32),
        pltpu.SemaphoreType.DMA,
    ],
)


def _add_body(a_ref, b_ref, o_ref):
    o_ref[...] = a_ref[...] + b_ref[...]


_combine = pl.pallas_call(
    _add_body,
    out_shape=jax.ShapeDtypeStruct((N_PAD, D_FEAT), jnp.float32),
    grid=(8,),
    in_specs=[
        pl.BlockSpec((N_PAD // 8, D_FEAT), lambda i: (i, 0)),
        pl.BlockSpec((N_PAD // 8, D_FEAT), lambda i: (i, 0)),
    ],
    out_specs=pl.BlockSpec((N_PAD // 8, D_FEAT), lambda i: (i, 0)),
)


@jax.jit
def kernel(edge_index, edge_vals, embeds):
    ei = edge_index.astype(jnp.int32)
    npad = E_PAD - N_EDGES
    dst = jnp.pad(ei[0], (0, npad)).reshape(IDX_ROWS, SUB)
    src = jnp.pad(ei[1], (0, npad)).reshape(IDX_ROWS, SUB)
    vals = jnp.pad(edge_vals, (0, npad))
    parts = _sc_spmm(src, dst, vals, embeds)
    return _combine(parts[0], parts[1])[:N_NODES]


# speedup vs baseline: 3.5527x; 1.1726x over previous
import jax
import jax.numpy as jnp
from jax import lax
from jax.experimental import pallas as pl
from jax.experimental.pallas import tpu as pltpu
from jax.experimental.pallas import tpu_sc as plsc

N_NODES = 10000
N_EDGES = 320000
D_FEAT = 128
NLANE = D_FEAT // 16

NC = 2
NS = 16
NW = NC * NS

N_PAD = 10240
SUB = 32
SPC = 8
CH = SUB * SPC
NCH = 40
EPW = CH * NCH
E_PAD = EPW * NW
IDX_ROWS = E_PAD // SUB
RPW = EPW // SUB
ZROWS = 16
RPT = N_PAD // NS


def _sc_body(src_r, dst_r, vals, embeds, parts,
             acc, src_buf, dst_buf, val_buf, rows_buf, zero_buf,
             gsem0, gsem1, ssem, isem):
    c = lax.axis_index("c")
    s = lax.axis_index("s")
    wid = s * NC + c

    zvec = jnp.zeros((16,), jnp.float32)
    for r in range(ZROWS):
        for d in range(NLANE):
            zero_buf[r, pl.ds(d * 16, 16)] = zvec
    for i in range(RPT // ZROWS):
        pltpu.sync_copy(zero_buf, acc.at[pl.ds(s * RPT + i * ZROWS, ZROWS)])
    plsc.subcore_barrier()

    row0 = wid * RPW
    e0 = wid * EPW

    gsems = (gsem0, gsem1)

    def chunk_body(i, carry):
        rb = row0 + i * SPC
        eb = e0 + i * CH
        idx_cps = [
            pltpu.async_copy(src_r.at[pl.ds(rb, SPC)], src_buf, isem),
            pltpu.async_copy(dst_r.at[pl.ds(rb, SPC)], dst_buf, isem),
            pltpu.async_copy(vals.at[pl.ds(eb, CH)], val_buf, isem),
        ]
        for cp in idx_cps:
            cp.wait()

        gcps = [None] * SPC
        gcps[0] = pltpu.async_copy(
            embeds.at[src_buf.at[0]], rows_buf.at[0], gsems[0]
        )
        scps = []
        for j in range(SPC):
            if j + 1 < SPC:
                gcps[j + 1] = pltpu.async_copy(
                    embeds.at[src_buf.at[j + 1]],
                    rows_buf.at[j + 1],
                    gsems[(j + 1) % 2],
                )
            gcps[j].wait()

            def mul_body(g, carry2, j=j):
                off = pl.multiple_of(g * 16, 16)
                v16 = val_buf[pl.ds(j * SUB + off, 16)]
                for k in range(16):
                    bv = jnp.full((16,), v16[k])
                    r = off + k
                    for d in range(NLANE):
                        rows_buf[j, r, pl.ds(d * 16, 16)] = (
                            rows_buf[j, r, pl.ds(d * 16, 16)] * bv
                        )
                return carry2

            lax.fori_loop(0, SUB // 16, mul_body, 0)
            scps.append(
                pltpu.async_copy(
                    rows_buf.at[j], acc.at[dst_buf.at[j]], ssem, add=True
                )
            )
        for cp in scps:
            cp.wait()
        return carry

    lax.fori_loop(0, NCH, chunk_body, 0)
    plsc.subcore_barrier()
    pltpu.sync_copy(
        acc.at[pl.ds(s * RPT, RPT)],
        parts.at[c, pl.ds(s * RPT, RPT)],
    )


_sc_spmm = pl.kernel(
    _sc_body,
    out_type=jax.ShapeDtypeStruct((NC, N_PAD, D_FEAT), jnp.float32),
    mesh=plsc.VectorSubcoreMesh(core_axis_name="c", subcore_axis_name="s"),
    scratch_types=[
        pltpu.VMEM_SHARED((N_PAD, D_FEAT), jnp.float32),
        pltpu.VMEM((SPC, SUB), jnp.int32),
        pltpu.VMEM((SPC, SUB), jnp.int32),
        pltpu.VMEM((CH,), jnp.float32),
        pltpu.VMEM((SPC, SUB, D_FEAT), jnp.float32),
        pltpu.VMEM((ZROWS, D_FEAT), jnp.float32),
        pltpu.SemaphoreType.DMA,
        pltpu.SemaphoreType.DMA,
        pltpu.SemaphoreType.DMA,
        pltpu.SemaphoreType.DMA,
    ],
)


def _add_body(a_ref, b_ref, o_ref):
    o_ref[...] = a_ref[...] + b_ref[...]


_combine = pl.pallas_call(
    _add_body,
    out_shape=jax.ShapeDtypeStruct((N_PAD, D_FEAT), jnp.float32),
    grid=(8,),
    in_specs=[
        pl.BlockSpec((N_PAD // 8, D_FEAT), lambda i: (i, 0)),
        pl.BlockSpec((N_PAD // 8, D_FEAT), lambda i: (i, 0)),
    ],
    out_specs=pl.BlockSpec((N_PAD // 8, D_FEAT), lambda i: (i, 0)),
)


@jax.jit
def kernel(edge_index, edge_vals, embeds):
    ei = edge_index.astype(jnp.int32)
    npad = E_PAD - N_EDGES
    dst = jnp.pad(ei[0], (0, npad)).reshape(IDX_ROWS, SUB)
    src = jnp.pad(ei[1], (0, npad)).reshape(IDX_ROWS, SUB)
    vals = jnp.pad(edge_vals, (0, npad))
    parts = _sc_spmm(src, dst, vals, embeds)
    return _combine(parts[0], parts[1])[:N_NODES]


# speedup vs baseline: 6.4573x; 1.8176x over previous
import jax
import jax.numpy as jnp
from jax import lax
from jax.experimental import pallas as pl
from jax.experimental.pallas import tpu as pltpu
from jax.experimental.pallas import tpu_sc as plsc

N_NODES = 10000
N_EDGES = 320000
D_FEAT = 128
DH = D_FEAT // 2
NDH = DH // 16

NC = 2
NS = 16
NW = NC * NS

N_PAD = 10240
SUB = 32
SPC = 8
CH = SUB * SPC
NCH = 40
EPW = CH * NCH
E_PAD = EPW * NW
IDX_ROWS = E_PAD // SUB
RPW = EPW // SUB
ZROWS = 32
RPT = N_PAD // NS


def _sc_body(src_r, dst_r, vals, embh, parts,
             emb_s, acc_s, src_buf, dst_buf, val_buf, rows_buf, zero_buf,
             gsem0, gsem1, ssem, isem):
    c = lax.axis_index("c")
    s = lax.axis_index("s")
    wid = s * NC + c
    gsems = (gsem0, gsem1)

    zvec = jnp.zeros((16,), jnp.float32)
    for r in range(ZROWS):
        for d in range(NDH):
            zero_buf[r, pl.ds(d * 16, 16)] = zvec

    row0 = wid * RPW
    e0 = wid * EPW

    for h in range(2):
        for i in range(RPT // ZROWS):
            pltpu.sync_copy(
                zero_buf, acc_s.at[pl.ds(s * RPT + i * ZROWS, ZROWS)]
            )
        pltpu.sync_copy(
            embh.at[h, pl.ds(s * RPT, RPT)],
            emb_s.at[pl.ds(s * RPT, RPT)],
        )
        plsc.subcore_barrier()

        def chunk_body(i, carry):
            rb = row0 + i * SPC
            eb = e0 + i * CH
            idx_cps = [
                pltpu.async_copy(src_r.at[pl.ds(rb, SPC)], src_buf, isem),
                pltpu.async_copy(dst_r.at[pl.ds(rb, SPC)], dst_buf, isem),
                pltpu.async_copy(vals.at[pl.ds(eb, CH)], val_buf, isem),
            ]
            for cp in idx_cps:
                cp.wait()

            gcps = [None] * SPC
            gcps[0] = pltpu.async_copy(
                emb_s.at[src_buf.at[0]], rows_buf.at[0], gsems[0]
            )
            scps = []
            for j in range(SPC):
                if j + 1 < SPC:
                    gcps[j + 1] = pltpu.async_copy(
                        emb_s.at[src_buf.at[j + 1]],
                        rows_buf.at[j + 1],
                        gsems[(j + 1) % 2],
                    )
                gcps[j].wait()

                def mul_body(g, carry2, j=j):
                    off = pl.multiple_of(g * 16, 16)
                    v16 = val_buf[pl.ds(j * SUB + off, 16)]
                    for k in range(16):
                        bv = jnp.full((16,), v16[k])
                        r = off + k
                        for d in range(NDH):
                            rows_buf[j, r, pl.ds(d * 16, 16)] = (
                                rows_buf[j, r, pl.ds(d * 16, 16)] * bv
                            )
                    return carry2

                lax.fori_loop(0, SUB // 16, mul_body, 0)
                scps.append(
                    pltpu.async_copy(
                        rows_buf.at[j], acc_s.at[dst_buf.at[j]], ssem,
                        add=True,
                    )
                )
            for cp in scps:
                cp.wait()
            return carry

        lax.fori_loop(0, NCH, chunk_body, 0)
        plsc.subcore_barrier()
        pltpu.sync_copy(
            acc_s.at[pl.ds(s * RPT, RPT)],
            parts.at[c, h, pl.ds(s * RPT, RPT)],
        )
        if h == 0:
            plsc.subcore_barrier()


_sc_spmm = pl.kernel(
    _sc_body,
    out_type=jax.ShapeDtypeStruct((NC, 2, N_PAD, DH), jnp.float32),
    mesh=plsc.VectorSubcoreMesh(core_axis_name="c", subcore_axis_name="s"),
    compiler_params=pltpu.CompilerParams(use_tc_tiling_on_sc=False),
    scratch_types=[
        pltpu.VMEM_SHARED((N_PAD, DH), jnp.float32),
        pltpu.VMEM_SHARED((N_PAD, DH), jnp.float32),
        pltpu.VMEM((SPC, SUB), jnp.int32),
        pltpu.VMEM((SPC, SUB), jnp.int32),
        pltpu.VMEM((CH,), jnp.float32),
        pltpu.VMEM((SPC, SUB, DH), jnp.float32),
        pltpu.VMEM((ZROWS, DH), jnp.float32),
        pltpu.SemaphoreType.DMA,
        pltpu.SemaphoreType.DMA,
        pltpu.SemaphoreType.DMA,
        pltpu.SemaphoreType.DMA,
    ],
)


def _add_body(a_ref, b_ref, o_ref):
    o_ref[...] = a_ref[...] + b_ref[...]


_combine = pl.pallas_call(
    _add_body,
    out_shape=jax.ShapeDtypeStruct((2 * N_PAD, DH), jnp.float32),
    grid=(5,),
    in_specs=[
        pl.BlockSpec((2 * N_PAD // 5, DH), lambda i: (i, 0)),
        pl.BlockSpec((2 * N_PAD // 5, DH), lambda i: (i, 0)),
    ],
    out_specs=pl.BlockSpec((2 * N_PAD // 5, DH), lambda i: (i, 0)),
)


@jax.jit
def kernel(edge_index, edge_vals, embeds):
    ei = edge_index.astype(jnp.int32)
    npad = E_PAD - N_EDGES
    dst = jnp.pad(ei[0], (0, npad)).reshape(IDX_ROWS, SUB)
    src = jnp.pad(ei[1], (0, npad)).reshape(IDX_ROWS, SUB)
    vals = jnp.pad(edge_vals, (0, npad))
    embh = (
        jnp.pad(embeds, ((0, N_PAD - N_NODES), (0, 0)))
        .reshape(N_PAD, 2, DH)
        .transpose(1, 0, 2)
    )
    parts = _sc_spmm(src, dst, vals, embh)
    summed = _combine(
        parts[0].reshape(2 * N_PAD, DH), parts[1].reshape(2 * N_PAD, DH)
    )
    return (
        summed.reshape(2, N_PAD, DH)
        .transpose(1, 0, 2)
        .reshape(N_PAD, D_FEAT)[:N_NODES]
    )


# speedup vs baseline: 6.4750x; 1.0027x over previous
import jax
import jax.numpy as jnp
from jax import lax
from jax.experimental import pallas as pl
from jax.experimental.pallas import tpu as pltpu
from jax.experimental.pallas import tpu_sc as plsc

N_NODES = 10000
N_EDGES = 320000
D_FEAT = 128
DH = D_FEAT // 2
NDH = DH // 16

NC = 2
NS = 16
NW = NC * NS

N_PAD = 10240
SUB = 32
SPC = 8
CH = SUB * SPC
NCH = 40
EPW = CH * NCH
E_PAD = EPW * NW
IDX_ROWS = E_PAD // SUB
RPW = EPW // SUB
ZROWS = 32
RPT = N_PAD // NS


def _sc_body(src_r, dst_r, vals, embh, parts,
             emb_s, acc_s, src_buf, dst_buf, val_buf, rows_buf, zero_buf,
             gsem0, gsem1, ssem, isem):
    c = lax.axis_index("c")
    s = lax.axis_index("s")
    wid = s * NC + c
    gsems = (gsem0, gsem1)

    zvec = jnp.zeros((16,), jnp.float32)
    for r in range(ZROWS):
        for d in range(NDH):
            zero_buf[r, pl.ds(d * 16, 16)] = zvec

    row0 = wid * RPW
    e0 = wid * EPW

    for h in range(2):
        for i in range(RPT // ZROWS):
            pltpu.sync_copy(
                zero_buf, acc_s.at[pl.ds(s * RPT + i * ZROWS, ZROWS)]
            )
        pltpu.sync_copy(
            embh.at[h, pl.ds(s * RPT, RPT)],
            emb_s.at[pl.ds(s * RPT, RPT)],
        )
        plsc.subcore_barrier()

        def chunk_body(i, carry):
            rb = row0 + i * SPC
            eb = e0 + i * CH
            idx_cps = [
                pltpu.async_copy(src_r.at[pl.ds(rb, SPC)], src_buf, isem),
                pltpu.async_copy(dst_r.at[pl.ds(rb, SPC)], dst_buf, isem),
                pltpu.async_copy(vals.at[pl.ds(eb, CH)], val_buf, isem),
            ]
            for cp in idx_cps:
                cp.wait()

            gcps = [None] * SPC
            gcps[0] = pltpu.async_copy(
                emb_s.at[src_buf.at[0]], rows_buf.at[0], gsems[0]
            )
            scps = []
            for j in range(SPC):
                if j + 1 < SPC:
                    gcps[j + 1] = pltpu.async_copy(
                        emb_s.at[src_buf.at[j + 1]],
                        rows_buf.at[j + 1],
                        gsems[(j + 1) % 2],
                    )
                gcps[j].wait()

                def mul_body(g, carry2, j=j):
                    off = pl.multiple_of(g * 16, 16)
                    v16 = val_buf[pl.ds(j * SUB + off, 16)]
                    for k in range(16):
                        bv = jnp.full((16,), v16[k])
                        r = off + k
                        for d in range(NDH):
                            rows_buf[j, r, pl.ds(d * 16, 16)] = (
                                rows_buf[j, r, pl.ds(d * 16, 16)] * bv
                            )
                    return carry2

                lax.fori_loop(0, SUB // 16, mul_body, 0)
                scps.append(
                    pltpu.async_copy(
                        rows_buf.at[j], acc_s.at[dst_buf.at[j]], ssem,
                        add=True,
                    )
                )
            for cp in scps:
                cp.wait()
            return carry

        lax.fori_loop(0, NCH, chunk_body, 0)
        plsc.subcore_barrier()
        pltpu.sync_copy(
            acc_s.at[pl.ds(s * RPT, RPT)],
            parts.at[c, h, pl.ds(s * RPT, RPT)],
        )
        if h == 0:
            plsc.subcore_barrier()


_sc_spmm = pl.kernel(
    _sc_body,
    out_type=jax.ShapeDtypeStruct((NC, 2, N_PAD, DH), jnp.float32),
    mesh=plsc.VectorSubcoreMesh(core_axis_name="c", subcore_axis_name="s"),
    compiler_params=pltpu.CompilerParams(use_tc_tiling_on_sc=False),
    scratch_types=[
        pltpu.VMEM_SHARED((N_PAD, DH), jnp.float32),
        pltpu.VMEM_SHARED((N_PAD, DH), jnp.float32),
        pltpu.VMEM((SPC, SUB), jnp.int32),
        pltpu.VMEM((SPC, SUB), jnp.int32),
        pltpu.VMEM((CH,), jnp.float32),
        pltpu.VMEM((SPC, SUB, DH), jnp.float32),
        pltpu.VMEM((ZROWS, DH), jnp.float32),
        pltpu.SemaphoreType.DMA,
        pltpu.SemaphoreType.DMA,
        pltpu.SemaphoreType.DMA,
        pltpu.SemaphoreType.DMA,
    ],
)


def _add_body(a0_ref, a1_ref, b0_ref, b1_ref, o_ref):
    o_ref[:, 0, :] = a0_ref[0, 0] + b0_ref[0, 0]
    o_ref[:, 1, :] = a1_ref[0, 0] + b1_ref[0, 0]


_RB = N_PAD // 8

_combine = pl.pallas_call(
    _add_body,
    out_shape=jax.ShapeDtypeStruct((N_PAD, 2, DH), jnp.float32),
    grid=(8,),
    in_specs=[
        pl.BlockSpec((1, 1, _RB, DH), lambda i: (0, 0, i, 0)),
        pl.BlockSpec((1, 1, _RB, DH), lambda i: (0, 1, i, 0)),
        pl.BlockSpec((1, 1, _RB, DH), lambda i: (1, 0, i, 0)),
        pl.BlockSpec((1, 1, _RB, DH), lambda i: (1, 1, i, 0)),
    ],
    out_specs=pl.BlockSpec((_RB, 2, DH), lambda i: (i, 0, 0)),
)


@jax.jit
def kernel(edge_index, edge_vals, embeds):
    ei = edge_index.astype(jnp.int32)
    npad = E_PAD - N_EDGES
    dst = jnp.pad(ei[0], (0, npad)).reshape(IDX_ROWS, SUB)
    src = jnp.pad(ei[1], (0, npad)).reshape(IDX_ROWS, SUB)
    vals = jnp.pad(edge_vals, (0, npad))
    embh = (
        jnp.pad(embeds, ((0, N_PAD - N_NODES), (0, 0)))
        .reshape(N_PAD, 2, DH)
        .transpose(1, 0, 2)
    )
    parts = _sc_spmm(src, dst, vals, embh)
    combined = _combine(parts, parts, parts, parts)
    return combined.reshape(N_PAD, D_FEAT)[:N_NODES]


# speedup vs baseline: 6.9163x; 1.0682x over previous
import jax
import jax.numpy as jnp
import numpy as np
from jax import lax
from jax.experimental import pallas as pl
from jax.experimental.pallas import tpu as pltpu
from jax.experimental.pallas import tpu_sc as plsc

N_NODES = 10000
N_EDGES = 320000
D_FEAT = 128
DH = D_FEAT // 2
NDH = DH // 16

NC = 2
NS = 16
NW = NC * NS

N_PAD = 10240
SUB = 32
SPC = 8
CH = SUB * SPC
NCH = 40
EPW = CH * NCH
E_PAD = EPW * NW
IDX_ROWS = E_PAD // SUB
RPW = EPW // SUB
ZROWS = 32
RPT = N_PAD // NS
IB = 4

_GDN = lax.GatherDimensionNumbers(
    offset_dims=(), collapsed_slice_dims=(0,), start_index_map=(0,)
)


def _bcast_lane(v16, k):
    idx = jnp.full((16, 1), k, jnp.int32)
    return lax.gather(
        v16, idx, _GDN, (1,),
        mode=lax.GatherScatterMode.PROMISE_IN_BOUNDS,
    )


def _sc_body(src_r, dst_r, vals, embh, parts,
             emb_s, acc_s, src_buf, dst_buf, val_buf, rows_buf, zero_buf,
             gsem0, gsem1, ssem, isem):
    c = lax.axis_index("c")
    s = lax.axis_index("s")
    wid = s * NC + c
    gsems = (gsem0, gsem1)

    zvec = jnp.zeros((16,), jnp.float32)
    for r in range(ZROWS):
        for d in range(NDH):
            zero_buf[r, pl.ds(d * 16, 16)] = zvec

    row0 = wid * RPW
    e0 = wid * EPW

    for h in range(2):
        for i in range(RPT // ZROWS):
            pltpu.sync_copy(
                zero_buf, acc_s.at[pl.ds(s * RPT + i * ZROWS, ZROWS)]
            )
        pltpu.sync_copy(
            embh.at[h, pl.ds(s * RPT, RPT)],
            emb_s.at[pl.ds(s * RPT, RPT)],
        )
        plsc.subcore_barrier()

        def chunk_body(i, carry):
            qq = lax.rem(i, IB)

            @pl.when(qq == 0)
            def _():
                rb = row0 + i * SPC
                eb = e0 + i * CH
                idx_cps = [
                    pltpu.async_copy(
                        src_r.at[pl.ds(rb, IB * SPC)], src_buf, isem
                    ),
                    pltpu.async_copy(
                        dst_r.at[pl.ds(rb, IB * SPC)], dst_buf, isem
                    ),
                    pltpu.async_copy(
                        vals.at[pl.ds(eb, IB * CH)], val_buf, isem
                    ),
                ]
                for cp in idx_cps:
                    cp.wait()

            qr = qq * SPC
            gcps = [None] * SPC
            gcps[0] = pltpu.async_copy(
                emb_s.at[src_buf.at[qr]], rows_buf.at[0], gsems[0]
            )
            scps = []
            for j in range(SPC):
                if j + 1 < SPC:
                    gcps[j + 1] = pltpu.async_copy(
                        emb_s.at[src_buf.at[qr + j + 1]],
                        rows_buf.at[j + 1],
                        gsems[(j + 1) % 2],
                    )
                gcps[j].wait()

                def mul_body(g, carry2, j=j):
                    off = pl.multiple_of(g * 16, 16)
                    v16 = val_buf[pl.ds(qq * CH + j * SUB + off, 16)]
                    for k in range(16):
                        bv = _bcast_lane(v16, k)
                        r = off + k
                        for d in range(NDH):
                            rows_buf[j, r, pl.ds(d * 16, 16)] = (
                                rows_buf[j, r, pl.ds(d * 16, 16)] * bv
                            )
                    return carry2

                lax.fori_loop(0, SUB // 16, mul_body, 0)
                scps.append(
                    pltpu.async_copy(
                        rows_buf.at[j], acc_s.at[dst_buf.at[qr + j]], ssem,
                        add=True,
                    )
                )
            for cp in scps:
                cp.wait()
            return carry

        lax.fori_loop(0, NCH, chunk_body, 0)
        plsc.subcore_barrier()
        pltpu.sync_copy(
            acc_s.at[pl.ds(s * RPT, RPT)],
            parts.at[c, h, pl.ds(s * RPT, RPT)],
        )
        if h == 0:
            plsc.subcore_barrier()


_sc_spmm = pl.kernel(
    _sc_body,
    out_type=jax.ShapeDtypeStruct((NC, 2, N_PAD, DH), jnp.float32),
    mesh=plsc.VectorSubcoreMesh(core_axis_name="c", subcore_axis_name="s"),
    compiler_params=pltpu.CompilerParams(use_tc_tiling_on_sc=False),
    scratch_types=[
        pltpu.VMEM_SHARED((N_PAD, DH), jnp.float32),
        pltpu.VMEM_SHARED((N_PAD, DH), jnp.float32),
        pltpu.VMEM((IB * SPC, SUB), jnp.int32),
        pltpu.VMEM((IB * SPC, SUB), jnp.int32),
        pltpu.VMEM((IB * CH,), jnp.float32),
        pltpu.VMEM((SPC, SUB, DH), jnp.float32),
        pltpu.VMEM((ZROWS, DH), jnp.float32),
        pltpu.SemaphoreType.DMA,
        pltpu.SemaphoreType.DMA,
        pltpu.SemaphoreType.DMA,
        pltpu.SemaphoreType.DMA,
    ],
)


def _add_body(a0_ref, a1_ref, b0_ref, b1_ref, o_ref):
    o_ref[:, 0, :] = a0_ref[0, 0] + b0_ref[0, 0]
    o_ref[:, 1, :] = a1_ref[0, 0] + b1_ref[0, 0]


_RB = N_PAD // 8

_combine = pl.pallas_call(
    _add_body,
    out_shape=jax.ShapeDtypeStruct((N_PAD, 2, DH), jnp.float32),
    grid=(8,),
    in_specs=[
        pl.BlockSpec((1, 1, _RB, DH), lambda i: (0, 0, i, 0)),
        pl.BlockSpec((1, 1, _RB, DH), lambda i: (0, 1, i, 0)),
        pl.BlockSpec((1, 1, _RB, DH), lambda i: (1, 0, i, 0)),
        pl.BlockSpec((1, 1, _RB, DH), lambda i: (1, 1, i, 0)),
    ],
    out_specs=pl.BlockSpec((_RB, 2, DH), lambda i: (i, 0, 0)),
)


@jax.jit
def kernel(edge_index, edge_vals, embeds):
    ei = edge_index.astype(jnp.int32)
    npad = E_PAD - N_EDGES
    dst = jnp.pad(ei[0], (0, npad)).reshape(IDX_ROWS, SUB)
    src = jnp.pad(ei[1], (0, npad)).reshape(IDX_ROWS, SUB)
    vals = jnp.pad(edge_vals, (0, npad))
    embh = (
        jnp.pad(embeds, ((0, N_PAD - N_NODES), (0, 0)))
        .reshape(N_PAD, 2, DH)
        .transpose(1, 0, 2)
    )
    parts = _sc_spmm(src, dst, vals, embh)
    combined = _combine(parts, parts, parts, parts)
    return combined.reshape(N_PAD, D_FEAT)[:N_NODES]


# speedup vs baseline: 6.9725x; 1.0081x over previous
import jax
import jax.numpy as jnp
import numpy as np
from jax import lax
from jax.experimental import pallas as pl
from jax.experimental.pallas import tpu as pltpu
from jax.experimental.pallas import tpu_sc as plsc

N_NODES = 10000
N_EDGES = 320000
D_FEAT = 128
DH = D_FEAT // 2
NDH = DH // 16

NC = 2
NS = 16
NW = NC * NS

N_PAD = 10240
SUB = 32
SPC = 8
CH = SUB * SPC
NCH = 40
EPW = CH * NCH
E_PAD = EPW * NW
IDX_ROWS = E_PAD // SUB
RPW = EPW // SUB
ZROWS = 32
RPT = N_PAD // NS
IB = 8

_GDN = lax.GatherDimensionNumbers(
    offset_dims=(), collapsed_slice_dims=(0,), start_index_map=(0,)
)


def _bcast_lane(v16, k):
    idx = jnp.full((16, 1), k, jnp.int32)
    return lax.gather(
        v16, idx, _GDN, (1,),
        mode=lax.GatherScatterMode.PROMISE_IN_BOUNDS,
    )


def _sc_body(src_r, dst_r, vals, embh, parts,
             emb_s, acc_s, src_buf, dst_buf, val_buf, rows_buf, zero_buf,
             gsem0, gsem1, ssem, isem):
    c = lax.axis_index("c")
    s = lax.axis_index("s")
    wid = s * NC + c
    gsems = (gsem0, gsem1)

    zvec = jnp.zeros((16,), jnp.float32)
    for r in range(ZROWS):
        for d in range(NDH):
            zero_buf[r, pl.ds(d * 16, 16)] = zvec

    row0 = wid * RPW
    e0 = wid * EPW

    for h in range(2):
        for i in range(RPT // ZROWS):
            pltpu.sync_copy(
                zero_buf, acc_s.at[pl.ds(s * RPT + i * ZROWS, ZROWS)]
            )
        pltpu.sync_copy(
            embh.at[h, pl.ds(s * RPT, RPT)],
            emb_s.at[pl.ds(s * RPT, RPT)],
        )
        plsc.subcore_barrier()

        def chunk_body(i, carry):
            qq = lax.rem(i, IB)

            @pl.when(qq == 0)
            def _():
                rb = row0 + i * SPC
                eb = e0 + i * CH
                idx_cps = [
                    pltpu.async_copy(
                        src_r.at[pl.ds(rb, IB * SPC)], src_buf, isem
                    ),
                    pltpu.async_copy(
                        dst_r.at[pl.ds(rb, IB * SPC)], dst_buf, isem
                    ),
                    pltpu.async_copy(
                        vals.at[pl.ds(eb, IB * CH)], val_buf, isem
                    ),
                ]
                for cp in idx_cps:
                    cp.wait()

            qr = qq * SPC
            gcps = [None] * SPC
            gcps[0] = pltpu.async_copy(
                emb_s.at[src_buf.at[qr]], rows_buf.at[0], gsems[0]
            )
            scps = []
            for j in range(SPC):
                if j + 1 < SPC:
                    gcps[j + 1] = pltpu.async_copy(
                        emb_s.at[src_buf.at[qr + j + 1]],
                        rows_buf.at[j + 1],
                        gsems[(j + 1) % 2],
                    )
                gcps[j].wait()

                def mul_body(g, carry2, j=j):
                    off = pl.multiple_of(g * 16, 16)
                    v16 = val_buf[pl.ds(qq * CH + j * SUB + off, 16)]
                    for k in range(16):
                        bv = _bcast_lane(v16, k)
                        r = off + k
                        for d in range(NDH):
                            rows_buf[j, r, pl.ds(d * 16, 16)] = (
                                rows_buf[j, r, pl.ds(d * 16, 16)] * bv
                            )
                    return carry2

                lax.fori_loop(0, SUB // 16, mul_body, 0)
                scps.append(
                    pltpu.async_copy(
                        rows_buf.at[j], acc_s.at[dst_buf.at[qr + j]], ssem,
                        add=True,
                    )
                )
            for cp in scps:
                cp.wait()
            return carry

        lax.fori_loop(0, NCH, chunk_body, 0)
        plsc.subcore_barrier()
        pltpu.sync_copy(
            acc_s.at[pl.ds(s * RPT, RPT)],
            parts.at[c, h, pl.ds(s * RPT, RPT)],
        )
        if h == 0:
            plsc.subcore_barrier()


_sc_spmm = pl.kernel(
    _sc_body,
    out_type=jax.ShapeDtypeStruct((NC, 2, N_PAD, DH), jnp.float32),
    mesh=plsc.VectorSubcoreMesh(core_axis_name="c", subcore_axis_name="s"),
    compiler_params=pltpu.CompilerParams(use_tc_tiling_on_sc=False),
    scratch_types=[
        pltpu.VMEM_SHARED((N_PAD, DH), jnp.float32),
        pltpu.VMEM_SHARED((N_PAD, DH), jnp.float32),
        pltpu.VMEM((IB * SPC, SUB), jnp.int32),
        pltpu.VMEM((IB * SPC, SUB), jnp.int32),
        pltpu.VMEM((IB * CH,), jnp.float32),
        pltpu.VMEM((SPC, SUB, DH), jnp.float32),
        pltpu.VMEM((ZROWS, DH), jnp.float32),
        pltpu.SemaphoreType.DMA,
        pltpu.SemaphoreType.DMA,
        pltpu.SemaphoreType.DMA,
        pltpu.SemaphoreType.DMA,
    ],
)


def _add_body(a0_ref, a1_ref, b0_ref, b1_ref, o_ref):
    o_ref[:, 0, :] = a0_ref[0, 0] + b0_ref[0, 0]
    o_ref[:, 1, :] = a1_ref[0, 0] + b1_ref[0, 0]


_RB = N_PAD // 8

_combine = pl.pallas_call(
    _add_body,
    out_shape=jax.ShapeDtypeStruct((N_PAD, 2, DH), jnp.float32),
    grid=(8,),
    in_specs=[
        pl.BlockSpec((1, 1, _RB, DH), lambda i: (0, 0, i, 0)),
        pl.BlockSpec((1, 1, _RB, DH), lambda i: (0, 1, i, 0)),
        pl.BlockSpec((1, 1, _RB, DH), lambda i: (1, 0, i, 0)),
        pl.BlockSpec((1, 1, _RB, DH), lambda i: (1, 1, i, 0)),
    ],
    out_specs=pl.BlockSpec((_RB, 2, DH), lambda i: (i, 0, 0)),
)


@jax.jit
def kernel(edge_index, edge_vals, embeds):
    ei = edge_index.astype(jnp.int32)
    npad = E_PAD - N_EDGES
    dst = jnp.pad(ei[0], (0, npad)).reshape(IDX_ROWS, SUB)
    src = jnp.pad(ei[1], (0, npad)).reshape(IDX_ROWS, SUB)
    vals = jnp.pad(edge_vals, (0, npad))
    embh = (
        jnp.pad(embeds, ((0, N_PAD - N_NODES), (0, 0)))
        .reshape(N_PAD, 2, DH)
        .transpose(1, 0, 2)
    )
    parts = _sc_spmm(src, dst, vals, embh)
    combined = _combine(parts, parts, parts, parts)
    return combined.reshape(N_PAD, D_FEAT)[:N_NODES]
